# baseline (speedup 1.0000x reference)
import functools

import jax
import jax.numpy as jnp
from jax import lax
from jax.experimental import pallas as pl
from jax.experimental.pallas import tpu as pltpu
from jax.experimental.pallas import tpu_sc as plsc

N = 10000
E = 320000
D = 128

NC = 2
NS = 16
DH = D // NC
K = 200
NBUF = 4
CH = 100
NG = CH // NBUF
EPT = CH * K
ACC_ROWS = 10240
ZR = ACC_ROWS // NS
RPT = N // NS


def _elu(x):
    return jnp.where(x > 0, x, jnp.exp(jnp.minimum(x, 0.0)) - 1.0)


def _dense_body(x_ref, w_ref, b_ref, o_ref):
    res = (
        jnp.dot(x_ref[...], w_ref[...], preferred_element_type=jnp.float32)
        + b_ref[...]
    )
    o_ref[...] = res.astype(jnp.bfloat16)


def _combine_dense_body(q_ref, w_ref, b_ref, o_ref):
    h = _elu(q_ref[...].astype(jnp.float32))
    res = jnp.dot(h, w_ref[...], preferred_element_type=jnp.float32) + b_ref[...]
    o_ref[...] = res.astype(jnp.bfloat16)


def _final_body(q_ref, o_ref):
    o_ref[...] = _elu(q_ref[...].astype(jnp.float32))


_ROWS_BLK = 2000
_GRID = N // _ROWS_BLK

_x_spec = pl.BlockSpec((_ROWS_BLK, D), lambda i: (i, 0))
_w_spec = pl.BlockSpec((D, D), lambda i: (0, 0))
_b_spec = pl.BlockSpec((1, D), lambda i: (0, 0))
_o_spec = pl.BlockSpec((_ROWS_BLK, D), lambda i: (i, 0))
_bf_shape = jax.ShapeDtypeStruct((N, D), jnp.bfloat16)


def _dense(x, w, b):
    return pl.pallas_call(
        _dense_body,
        grid=(_GRID,),
        in_specs=[_x_spec, _w_spec, _b_spec],
        out_specs=_o_spec,
        out_shape=_bf_shape,
    )(x, w, b.reshape(1, D))


def _combine_dense(q, w, b):
    return pl.pallas_call(
        _combine_dense_body,
        grid=(_GRID,),
        in_specs=[_x_spec, _w_spec, _b_spec],
        out_specs=_o_spec,
        out_shape=_bf_shape,
    )(q, w, b.reshape(1, D))


def _final(q):
    return pl.pallas_call(
        _final_body,
        grid=(_GRID,),
        in_specs=[_x_spec],
        out_specs=_o_spec,
        out_shape=jax.ShapeDtypeStruct((N, D), jnp.float32),
    )(q)


@functools.partial(
    pl.kernel,
    out_type=jax.ShapeDtypeStruct((ACC_ROWS, D), jnp.bfloat16),
    mesh=plsc.VectorSubcoreMesh(core_axis_name="c", subcore_axis_name="s"),
    compiler_params=pltpu.CompilerParams(use_tc_tiling_on_sc=False),
    scratch_types=[
        pltpu.VMEM((EPT,), jnp.int32),
        pltpu.VMEM((EPT,), jnp.int32),
        pltpu.VMEM((NBUF, K, DH), jnp.bfloat16),
        pltpu.VMEM_SHARED((N, DH), jnp.bfloat16),
        pltpu.VMEM_SHARED((ACC_ROWS, DH), jnp.bfloat16),
        pltpu.SemaphoreType.DMA,
        pltpu.SemaphoreType.DMA,
        pltpu.SemaphoreType.DMA,
        pltpu.SemaphoreType.DMA,
        pltpu.SemaphoreType.DMA,
        pltpu.SemaphoreType.DMA,
        pltpu.SemaphoreType.DMA,
        pltpu.SemaphoreType.DMA,
    ],
)
def _sc_agg(support_hbm, srcr_hbm, dstr_hbm, zeros_hbm, out_hbm,
            idx_s, idx_d, rbuf, sup, acc, g0, g1, g2, g3, s0, s1, s2, s3):
    gsems = [g0, g1, g2, g3]
    ssems = [s0, s1, s2, s3]
    c = lax.axis_index("c")
    s = lax.axis_index("s")

    pltpu.sync_copy(
        support_hbm.at[pl.ds(s * RPT, RPT), pl.ds(c * DH, DH)],
        sup.at[pl.ds(s * RPT, RPT)],
    )
    pltpu.sync_copy(zeros_hbm, acc.at[pl.ds(s * ZR, ZR)])
    pltpu.sync_copy(srcr_hbm.at[pl.ds(s * EPT, EPT)], idx_s)
    pltpu.sync_copy(dstr_hbm.at[pl.ds(s * EPT, EPT)], idx_d)
    plsc.subcore_barrier()

    def gather(j, b):
        pltpu.async_copy(sup.at[idx_s.at[pl.ds(j * K, K)]], rbuf.at[b],
                         gsems[b])

    def gather_wait(j, b):
        pltpu.make_async_copy(sup.at[idx_s.at[pl.ds(j * K, K)]], rbuf.at[b],
                              gsems[b]).wait()

    def scatter(j, b):
        pltpu.async_copy(rbuf.at[b], acc.at[idx_d.at[pl.ds(j * K, K)]],
                         ssems[b], add=True)

    def scatter_wait(j, b):
        pltpu.make_async_copy(rbuf.at[b], acc.at[idx_d.at[pl.ds(j * K, K)]],
                              ssems[b]).wait()

    for b in range(NBUF):
        gather(b, b)

    def group(t, carry):
        base = t * NBUF
        for b in range(NBUF):
            gather_wait(base + b, b)
            scatter(base + b, b)
        for b in range(NBUF):
            scatter_wait(base + b, b)
            gather(base + NBUF + b, b)
        return carry

    lax.fori_loop(0, NG - 1, group, 0)

    ebase = (NG - 1) * NBUF
    for b in range(NBUF):
        gather_wait(ebase + b, b)
        scatter(ebase + b, b)
    for b in range(NBUF):
        scatter_wait(ebase + b, b)
    plsc.subcore_barrier()

    pltpu.sync_copy(
        acc.at[pl.ds(s * ZR, ZR)],
        out_hbm.at[pl.ds(s * ZR, ZR), pl.ds(c * DH, DH)],
    )


def kernel(features, adj, W1, b1, W2, b2):
    src = adj[0]
    dst = adj[1]
    zeros = jnp.zeros((ZR, DH), jnp.bfloat16)

    s1 = _dense(features, W1, b1)
    p = _sc_agg(s1, src, dst, zeros)
    s2 = _combine_dense(p, W2, b2)
    q = _sc_agg(s2, src, dst, zeros)
    return _final(q)


# speedup vs baseline: 1.1251x; 1.1251x over previous
import functools

import jax
import jax.numpy as jnp
from jax import lax
from jax.experimental import pallas as pl
from jax.experimental.pallas import tpu as pltpu
from jax.experimental.pallas import tpu_sc as plsc

N = 10000
E = 320000
D = 128

NC = 2
NS = 16
DH = D // NC
K = 200
NBUF = 4
CH = 100
NG = CH // NBUF
EPT = CH * K
ACC_ROWS = 10240
ZR = ACC_ROWS // NS
RPT = N // NS


def _elu(x):
    return jnp.where(x > 0, x, jnp.exp(jnp.minimum(x, 0.0)) - 1.0)


def _dense_body(x_ref, w_ref, b_ref, o_ref):
    res = (
        jnp.dot(x_ref[...], w_ref[...], preferred_element_type=jnp.float32)
        + b_ref[...]
    )
    o_ref[...] = res.astype(jnp.bfloat16)


def _combine_dense_body(q_ref, w_ref, b_ref, o_ref):
    h = _elu(q_ref[...].astype(jnp.float32))
    res = jnp.dot(h, w_ref[...], preferred_element_type=jnp.float32) + b_ref[...]
    o_ref[...] = res.astype(jnp.bfloat16)


def _final_body(q_ref, o_ref):
    o_ref[...] = _elu(q_ref[...].astype(jnp.float32))


_ROWS_BLK = 2000
_GRID = N // _ROWS_BLK

_x_spec = pl.BlockSpec((_ROWS_BLK, D), lambda i: (i, 0))
_w_spec = pl.BlockSpec((D, D), lambda i: (0, 0))
_b_spec = pl.BlockSpec((1, D), lambda i: (0, 0))
_o_spec = pl.BlockSpec((_ROWS_BLK, D), lambda i: (i, 0))
_bf_shape = jax.ShapeDtypeStruct((N, D), jnp.bfloat16)


def _dense(x, w, b):
    return pl.pallas_call(
        _dense_body,
        grid=(_GRID,),
        in_specs=[_x_spec, _w_spec, _b_spec],
        out_specs=_o_spec,
        out_shape=_bf_shape,
    )(x, w, b.reshape(1, D))


def _combine_dense(q, w, b):
    return pl.pallas_call(
        _combine_dense_body,
        grid=(_GRID,),
        in_specs=[_x_spec, _w_spec, _b_spec],
        out_specs=_o_spec,
        out_shape=_bf_shape,
    )(q, w, b.reshape(1, D))


def _final(q):
    return pl.pallas_call(
        _final_body,
        grid=(_GRID,),
        in_specs=[_x_spec],
        out_specs=_o_spec,
        out_shape=jax.ShapeDtypeStruct((N, D), jnp.float32),
    )(q)


@functools.partial(
    pl.kernel,
    out_type=jax.ShapeDtypeStruct((ACC_ROWS, D), jnp.bfloat16),
    mesh=plsc.VectorSubcoreMesh(core_axis_name="c", subcore_axis_name="s"),
    compiler_params=pltpu.CompilerParams(use_tc_tiling_on_sc=False),
    scratch_types=[
        pltpu.VMEM((EPT,), jnp.int32),
        pltpu.VMEM((EPT,), jnp.int32),
        pltpu.VMEM((NBUF, K, DH), jnp.bfloat16),
        pltpu.VMEM_SHARED((N, DH), jnp.bfloat16),
        pltpu.VMEM_SHARED((ACC_ROWS, DH), jnp.bfloat16),
        pltpu.SemaphoreType.DMA,
        pltpu.SemaphoreType.DMA,
        pltpu.SemaphoreType.DMA,
        pltpu.SemaphoreType.DMA,
    ],
)
def _sc_agg(support_hbm, srcr_hbm, dstr_hbm, zeros_hbm, out_hbm,
            idx_s, idx_d, rbuf, sup, acc, g0, g1, g2, g3):
    gsems = [g0, g1, g2, g3]
    c = lax.axis_index("c")
    s = lax.axis_index("s")

    pltpu.sync_copy(
        support_hbm.at[pl.ds(s * RPT, RPT), pl.ds(c * DH, DH)],
        sup.at[pl.ds(s * RPT, RPT)],
    )
    pltpu.sync_copy(zeros_hbm, acc.at[pl.ds(s * ZR, ZR)])
    pltpu.sync_copy(srcr_hbm.at[pl.ds(s * EPT, EPT)], idx_s)
    pltpu.sync_copy(dstr_hbm.at[pl.ds(s * EPT, EPT)], idx_d)
    plsc.subcore_barrier()

    def gather(j, b):
        pltpu.async_copy(sup.at[idx_s.at[pl.ds(j * K, K)]], rbuf.at[b],
                         gsems[b])

    def gather_wait(j, b):
        pltpu.make_async_copy(sup.at[idx_s.at[pl.ds(j * K, K)]], rbuf.at[b],
                              gsems[b]).wait()

    def scatter(j, b):
        pltpu.sync_copy(rbuf.at[b], acc.at[idx_d.at[pl.ds(j * K, K)]],
                        add=True)

    for b in range(NBUF):
        gather(b, b)

    def group(t, carry):
        base = t * NBUF
        for b in range(NBUF):
            gather_wait(base + b, b)
            scatter(base + b, b)
            gather(base + NBUF + b, b)
        return carry

    lax.fori_loop(0, NG - 1, group, 0)

    ebase = (NG - 1) * NBUF
    for b in range(NBUF):
        gather_wait(ebase + b, b)
        scatter(ebase + b, b)
    plsc.subcore_barrier()

    pltpu.sync_copy(
        acc.at[pl.ds(s * ZR, ZR)],
        out_hbm.at[pl.ds(s * ZR, ZR), pl.ds(c * DH, DH)],
    )


def kernel(features, adj, W1, b1, W2, b2):
    src = adj[0]
    dst = adj[1]
    zeros = jnp.zeros((ZR, DH), jnp.bfloat16)

    s1 = _dense(features, W1, b1)
    p = _sc_agg(s1, src, dst, zeros)
    s2 = _combine_dense(p, W2, b2)
    q = _sc_agg(s2, src, dst, zeros)
    return _final(q)
